# baseline (speedup 1.0000x reference)
import functools

import jax
import jax.numpy as jnp
import numpy as np
from jax import lax
from jax.experimental import pallas as pl
from jax.experimental.pallas import tpu as pltpu
from jax.experimental.pallas import tpu_sc as plsc

EPS = 1e-3
NTILES = 32
CHUNK = 400

_ILV = np.empty((32,), np.int32)
_ILV[0::2] = np.arange(16)
_ILV[1::2] = np.arange(16, 32)


def _tc_pre_body(x_ref, wp_ref, bp_ref, wc_ref, bc_ref, c_ref):
    h = jnp.dot(x_ref[...], wp_ref[...], preferred_element_type=jnp.float32)
    h = jnp.maximum(h + bp_ref[...], 0.0)
    ab = jnp.dot(h, wc_ref[0], preferred_element_type=jnp.float32) + bc_ref[0]
    c_ref[...] = ab.astype(jnp.bfloat16)


def _tc_head_body(p_ref, wpost_ref, bpost_ref, wout_ref, bout_ref, o_ref):
    g = jnp.sum(p_ref[...], axis=0)
    t = jnp.dot(g, wpost_ref[...], preferred_element_type=jnp.float32)
    t = jnp.maximum(t + bpost_ref[...], 0.0)
    z = jnp.sum(t * wout_ref[...], axis=1, keepdims=True) + bout_ref[...]
    o_ref[...] = jax.nn.sigmoid(z)


def _sc_edge_kernel(n, e, h, g):
    ept = e // NTILES
    nchunks = ept // CHUNK
    nrows = 2 * CHUNK
    sub = 80

    mesh = plsc.VectorSubcoreMesh(core_axis_name="c", subcore_axis_name="s")

    assert nchunks % 2 == 1 and nrows % sub == 0 and CHUNK % 16 == 0

    @functools.partial(
        pl.kernel,
        out_type=jax.ShapeDtypeStruct((NTILES, g * h), jnp.float32),
        mesh=mesh,
        compiler_params=pltpu.CompilerParams(
            needs_layout_passes=False, use_tc_tiling_on_sc=False),
        scratch_types=[
            pltpu.VMEM((n,), jnp.int32),
            pltpu.VMEM((g * h,), jnp.float32),
            pltpu.VMEM((g * h,), jnp.float32),
            pltpu.VMEM((g * h,), jnp.float32),
            pltpu.VMEM((g * h,), jnp.float32),
            pltpu.VMEM((ept,), jnp.int32),
            pltpu.VMEM((ept,), jnp.int32),
            pltpu.VMEM((2 * ept,), jnp.int32),
            pltpu.VMEM((ept,), jnp.int32),
            pltpu.VMEM((nrows, h), jnp.bfloat16),
            pltpu.VMEM((nrows, h), jnp.bfloat16),
            pltpu.SemaphoreType.DMA,
            pltpu.SemaphoreType.DMA,
        ],
    )
    def body(dst_hbm, src_hbm, i_hbm, c_hbm, out_hbm,
             i_v, acc, acc1, acc2, acc3, dst_all, src_all, cidx, gid_all,
             cr0, cr1, s0, s1):
        banks = (acc, acc1, acc2, acc3)
        wid = lax.axis_index("c") * 16 + lax.axis_index("s")
        ebase = pl.multiple_of(wid * ept, 8)

        pltpu.sync_copy(i_hbm, i_v)
        pltpu.sync_copy(dst_hbm.at[pl.ds(ebase, ept)], dst_all)
        pltpu.sync_copy(src_hbm.at[pl.ds(ebase, ept)], src_all)

        def zero_body(k, _):
            z = jnp.zeros((16,), jnp.float32)
            acc[pl.ds(k * 16, 16)] = z
            acc1[pl.ds(k * 16, 16)] = z
            acc2[pl.ds(k * 16, 16)] = z
            acc3[pl.ds(k * 16, 16)] = z
            return _
        lax.fori_loop(0, (g * h) // 16, zero_body, None)

        iota = lax.iota(jnp.int32, 16)

        def prep_body(q, _):
            d16 = dst_all[pl.ds(q * 16, 16)]
            s16 = src_all[pl.ds(q * 16, 16)]
            gid_all[pl.ds(q * 16, 16)] = plsc.load_gather(i_v, (d16,))
            pos = q * 32 + iota * 2
            plsc.store_scatter(cidx, (pos,), d16)
            plsc.store_scatter(cidx, (pos + 1,), s16 + n)
            return _
        lax.fori_loop(0, ept // 16, prep_body, None)

        def issue(c, cr, sem):
            base = pl.multiple_of(c * nrows, 8)
            for s in range(nrows // sub):
                pltpu.async_copy(
                    c_hbm.at[cidx.at[pl.ds(base + s * sub, sub)]],
                    cr.at[pl.ds(s * sub, sub)], sem)

        def wait(cr, sem):
            for s in range(nrows // sub):
                pltpu.make_async_copy(
                    c_hbm.at[pl.ds(0, sub)],
                    cr.at[pl.ds(s * sub, sub)], sem).wait()

        def compute(c, cr):
            def group_body(q, _):
                gvec = gid_all[pl.ds(c * CHUNK + q * 16, 16)]
                for l in range(16):
                    ei = q * 16 + l
                    ge = gvec[l]
                    off = pl.multiple_of(ge * h, h)
                    am = cr[2 * ei, pl.ds(0, h)]
                    bm = cr[2 * ei + 1, pl.ds(0, h)]
                    m = jnp.maximum(am + bm, 0.0)
                    v0, v1 = plsc.unpack(
                        m, format=plsc.PackFormat.INTERLEAVED,
                        preferred_element_type=jnp.float32)
                    bank = banks[l % 4]
                    plsc.addupdate(bank.at[pl.ds(off, 16)], v0)
                    plsc.addupdate(bank.at[pl.ds(off + 16, 16)], v1)
                return _
            lax.fori_loop(0, CHUNK // 16, group_body, None)

        issue(0, cr0, s0)

        def pair_body(it, _):
            c = it * 2
            issue(c + 1, cr1, s1)
            wait(cr0, s0)
            compute(c, cr0)
            issue(c + 2, cr0, s0)
            wait(cr1, s1)
            compute(c + 1, cr1)
            return _
        lax.fori_loop(0, (nchunks - 1) // 2, pair_body, None)

        wait(cr0, s0)
        compute(nchunks - 1, cr0)

        def merge_body(k, _):
            s = pl.ds(k * 16, 16)
            acc[s] = (acc[s] + acc1[s]) + (acc2[s] + acc3[s])
            return _
        lax.fori_loop(0, (g * h) // 16, merge_body, None)
        pltpu.sync_copy(acc, out_hbm.at[wid])

    return body


def kernel(x, edge_index, i, W_pre, b_pre, gamma_pre, beta_pre, W_conv,
           b_conv, W_post, b_post, gamma_post, beta_post, W_out, b_out):
    n, d = x.shape
    e = edge_index.shape[1]
    h = W_pre.shape[1]
    g = 128
    assert e % (NTILES * CHUNK) == 0 and h == 32

    k = 1.0 / jnp.sqrt(1.0 + EPS)
    wp = W_pre * (gamma_pre * k)[None, :]
    bp = (b_pre * gamma_pre * k + beta_pre).reshape(1, h)
    w1 = W_conv[:h]
    w2 = W_conv[h:]
    ilv = jnp.asarray(_ILV)
    wc = jnp.stack([(w1 - w2)[:, ilv], w2[:, ilv]])
    bc = jnp.stack([b_conv[ilv], jnp.zeros_like(b_conv)]).reshape(2, 1, h)
    wpost = W_post * (gamma_post * k)[None, :]
    bpost = (b_post * gamma_post * k + beta_post).reshape(1, h)
    wout = W_out.reshape(1, h)
    bout = b_out.reshape(1, 1)

    rows = 1000
    c_nodes = pl.pallas_call(
        _tc_pre_body,
        grid=(2, n // rows),
        in_specs=[
            pl.BlockSpec((rows, d), lambda t, j: (j, 0)),
            pl.BlockSpec((d, h), lambda t, j: (0, 0)),
            pl.BlockSpec((1, h), lambda t, j: (0, 0)),
            pl.BlockSpec((1, h, h), lambda t, j: (t, 0, 0)),
            pl.BlockSpec((1, 1, h), lambda t, j: (t, 0, 0)),
        ],
        out_specs=pl.BlockSpec(
            (rows, h), lambda t, j, nb=n // rows: (t * nb + j, 0)),
        out_shape=jax.ShapeDtypeStruct((2 * n, h), jnp.bfloat16),
    )(x, wp, bp, wc, bc)

    src = edge_index[0]
    dst = edge_index[1]
    partials = _sc_edge_kernel(n, e, h, g)(dst, src, i, c_nodes)
    partials = partials.reshape(NTILES, g, h)

    out = pl.pallas_call(
        _tc_head_body,
        in_specs=[
            pl.BlockSpec((NTILES, g, h), lambda: (0, 0, 0)),
            pl.BlockSpec((h, h), lambda: (0, 0)),
            pl.BlockSpec((1, h), lambda: (0, 0)),
            pl.BlockSpec((1, h), lambda: (0, 0)),
            pl.BlockSpec((1, 1), lambda: (0, 0)),
        ],
        out_specs=pl.BlockSpec((g, 1), lambda: (0, 0)),
        out_shape=jax.ShapeDtypeStruct((g, 1), jnp.float32),
    )(partials, wpost, bpost, wout, bout)
    return out


# speedup vs baseline: 1.0010x; 1.0010x over previous
import functools

import jax
import jax.numpy as jnp
import numpy as np
from jax import lax
from jax.experimental import pallas as pl
from jax.experimental.pallas import tpu as pltpu
from jax.experimental.pallas import tpu_sc as plsc

EPS = 1e-3
NTILES = 32
CHUNK = 400

_ILV = np.empty((32,), np.int32)
_ILV[0::2] = np.arange(16)
_ILV[1::2] = np.arange(16, 32)


def _tc_pre_body(x_ref, wp_ref, bp_ref, wc_ref, bc_ref, c_ref):
    h = jnp.dot(x_ref[...], wp_ref[...], preferred_element_type=jnp.float32)
    h = jnp.maximum(h + bp_ref[...], 0.0)
    ab = jnp.dot(h, wc_ref[0], preferred_element_type=jnp.float32) + bc_ref[0]
    c_ref[...] = ab.astype(jnp.bfloat16)


def _tc_head_body(p_ref, wpost_ref, bpost_ref, wout_ref, bout_ref, o_ref):
    g = jnp.sum(p_ref[...], axis=0)
    t = jnp.dot(g, wpost_ref[...], preferred_element_type=jnp.float32)
    t = jnp.maximum(t + bpost_ref[...], 0.0)
    z = jnp.sum(t * wout_ref[...], axis=1, keepdims=True) + bout_ref[...]
    o_ref[...] = jax.nn.sigmoid(z)


def _sc_edge_kernel(n, e, h, g):
    ept = e // NTILES
    nchunks = ept // CHUNK
    nrows = 2 * CHUNK
    subs = []
    off = 0
    while off < nrows:
        sz = min(128, nrows - off)
        subs.append((off, sz))
        off += sz

    mesh = plsc.VectorSubcoreMesh(core_axis_name="c", subcore_axis_name="s")

    assert nchunks % 2 == 1 and CHUNK % 16 == 0
    assert all(o % 8 == 0 for o, _ in subs)

    @functools.partial(
        pl.kernel,
        out_type=jax.ShapeDtypeStruct((NTILES, g * h), jnp.float32),
        mesh=mesh,
        compiler_params=pltpu.CompilerParams(
            needs_layout_passes=False, use_tc_tiling_on_sc=False),
        scratch_types=[
            pltpu.VMEM((n,), jnp.int32),
            pltpu.VMEM((g * h,), jnp.float32),
            pltpu.VMEM((g * h,), jnp.float32),
            pltpu.VMEM((g * h,), jnp.float32),
            pltpu.VMEM((g * h,), jnp.float32),
            pltpu.VMEM((ept,), jnp.int32),
            pltpu.VMEM((ept,), jnp.int32),
            pltpu.VMEM((2 * ept,), jnp.int32),
            pltpu.VMEM((ept,), jnp.int32),
            pltpu.VMEM((nrows, h), jnp.bfloat16),
            pltpu.VMEM((nrows, h), jnp.bfloat16),
            pltpu.SemaphoreType.DMA,
            pltpu.SemaphoreType.DMA,
        ],
    )
    def body(dst_hbm, src_hbm, i_hbm, c_hbm, out_hbm,
             i_v, acc, acc1, acc2, acc3, dst_all, src_all, cidx, gid_all,
             cr0, cr1, s0, s1):
        banks = (acc, acc1, acc2, acc3)
        wid = lax.axis_index("c") * 16 + lax.axis_index("s")
        ebase = pl.multiple_of(wid * ept, 8)

        pltpu.sync_copy(i_hbm, i_v)
        pltpu.sync_copy(dst_hbm.at[pl.ds(ebase, ept)], dst_all)
        pltpu.sync_copy(src_hbm.at[pl.ds(ebase, ept)], src_all)

        def zero_body(k, _):
            z = jnp.zeros((16,), jnp.float32)
            acc[pl.ds(k * 16, 16)] = z
            acc1[pl.ds(k * 16, 16)] = z
            acc2[pl.ds(k * 16, 16)] = z
            acc3[pl.ds(k * 16, 16)] = z
            return _
        lax.fori_loop(0, (g * h) // 16, zero_body, None)

        iota = lax.iota(jnp.int32, 16)

        def prep_body(q, _):
            d16 = dst_all[pl.ds(q * 16, 16)]
            s16 = src_all[pl.ds(q * 16, 16)]
            gid_all[pl.ds(q * 16, 16)] = plsc.load_gather(i_v, (d16,))
            pos = q * 32 + iota * 2
            plsc.store_scatter(cidx, (pos,), d16)
            plsc.store_scatter(cidx, (pos + 1,), s16 + n)
            return _
        lax.fori_loop(0, ept // 16, prep_body, None)

        def issue(c, cr, sem):
            base = pl.multiple_of(c * nrows, 8)
            for o, sz in subs:
                pltpu.async_copy(
                    c_hbm.at[cidx.at[pl.ds(base + o, sz)]],
                    cr.at[pl.ds(o, sz)], sem)

        def wait(cr, sem):
            for o, sz in subs:
                pltpu.make_async_copy(
                    c_hbm.at[pl.ds(0, sz)],
                    cr.at[pl.ds(o, sz)], sem).wait()

        def compute(c, cr):
            def group_body(q, _):
                gvec = gid_all[pl.ds(c * CHUNK + q * 16, 16)]
                for l in range(16):
                    ei = q * 16 + l
                    ge = gvec[l]
                    off = pl.multiple_of(ge * h, h)
                    am = cr[2 * ei, pl.ds(0, h)]
                    bm = cr[2 * ei + 1, pl.ds(0, h)]
                    m = jnp.maximum(am + bm, 0.0)
                    v0, v1 = plsc.unpack(
                        m, format=plsc.PackFormat.INTERLEAVED,
                        preferred_element_type=jnp.float32)
                    bank = banks[l % 4]
                    plsc.addupdate(bank.at[pl.ds(off, 16)], v0)
                    plsc.addupdate(bank.at[pl.ds(off + 16, 16)], v1)
                return _
            lax.fori_loop(0, CHUNK // 16, group_body, None)

        issue(0, cr0, s0)

        def pair_body(it, _):
            c = it * 2
            issue(c + 1, cr1, s1)
            wait(cr0, s0)
            compute(c, cr0)
            issue(c + 2, cr0, s0)
            wait(cr1, s1)
            compute(c + 1, cr1)
            return _
        lax.fori_loop(0, (nchunks - 1) // 2, pair_body, None)

        wait(cr0, s0)
        compute(nchunks - 1, cr0)

        def merge_body(k, _):
            s = pl.ds(k * 16, 16)
            acc[s] = (acc[s] + acc1[s]) + (acc2[s] + acc3[s])
            return _
        lax.fori_loop(0, (g * h) // 16, merge_body, None)
        pltpu.sync_copy(acc, out_hbm.at[wid])

    return body


def kernel(x, edge_index, i, W_pre, b_pre, gamma_pre, beta_pre, W_conv,
           b_conv, W_post, b_post, gamma_post, beta_post, W_out, b_out):
    n, d = x.shape
    e = edge_index.shape[1]
    h = W_pre.shape[1]
    g = 128
    assert e % (NTILES * CHUNK) == 0 and h == 32

    k = 1.0 / jnp.sqrt(1.0 + EPS)
    wp = W_pre * (gamma_pre * k)[None, :]
    bp = (b_pre * gamma_pre * k + beta_pre).reshape(1, h)
    w1 = W_conv[:h]
    w2 = W_conv[h:]
    ilv = jnp.asarray(_ILV)
    wc = jnp.stack([(w1 - w2)[:, ilv], w2[:, ilv]])
    bc = jnp.stack([b_conv[ilv], jnp.zeros_like(b_conv)]).reshape(2, 1, h)
    wpost = W_post * (gamma_post * k)[None, :]
    bpost = (b_post * gamma_post * k + beta_post).reshape(1, h)
    wout = W_out.reshape(1, h)
    bout = b_out.reshape(1, 1)

    rows = 1000
    c_nodes = pl.pallas_call(
        _tc_pre_body,
        grid=(2, n // rows),
        in_specs=[
            pl.BlockSpec((rows, d), lambda t, j: (j, 0)),
            pl.BlockSpec((d, h), lambda t, j: (0, 0)),
            pl.BlockSpec((1, h), lambda t, j: (0, 0)),
            pl.BlockSpec((1, h, h), lambda t, j: (t, 0, 0)),
            pl.BlockSpec((1, 1, h), lambda t, j: (t, 0, 0)),
        ],
        out_specs=pl.BlockSpec(
            (rows, h), lambda t, j, nb=n // rows: (t * nb + j, 0)),
        out_shape=jax.ShapeDtypeStruct((2 * n, h), jnp.bfloat16),
    )(x, wp, bp, wc, bc)

    src = edge_index[0]
    dst = edge_index[1]
    partials = _sc_edge_kernel(n, e, h, g)(dst, src, i, c_nodes)
    partials = partials.reshape(NTILES, g, h)

    out = pl.pallas_call(
        _tc_head_body,
        in_specs=[
            pl.BlockSpec((NTILES, g, h), lambda: (0, 0, 0)),
            pl.BlockSpec((h, h), lambda: (0, 0)),
            pl.BlockSpec((1, h), lambda: (0, 0)),
            pl.BlockSpec((1, h), lambda: (0, 0)),
            pl.BlockSpec((1, 1), lambda: (0, 0)),
        ],
        out_specs=pl.BlockSpec((g, 1), lambda: (0, 0)),
        out_shape=jax.ShapeDtypeStruct((g, 1), jnp.float32),
    )(partials, wpost, bpost, wout, bout)
    return out
